# baseline (speedup 1.0000x reference)
import functools

import jax
import jax.numpy as jnp
from jax import lax
from jax.experimental import pallas as pl

NPG = 96
KNN = 16
FEA_IN = 5


def _relu(v):
    return jnp.maximum(v, 0.0)


def _leaky(v):
    return jnp.where(v >= 0, v, 0.01 * v)


def _conv_body(xx_ref, w11_ref, b11_ref, w21_ref, b21_ref,
               w12_ref, b12_ref, w22_ref, b22_ref,
               comb_ref, *, G):
    xx = xx_ref[...]

    d2 = jnp.zeros((G, NPG, NPG), jnp.float32)
    for c in range(3):
        pc = xx[:, :, 2 + c]
        diff = pc[:, :, None] - pc[:, None, :]
        d2 = d2 + diff * diff

    jidx = lax.broadcasted_iota(jnp.int32, (G, NPG, NPG), 2)
    work = d2
    oh_slots = []
    for _ in range(KNN):
        m = jnp.min(work, axis=-1, keepdims=True)
        eq = work == m
        selj = jnp.min(jnp.where(eq, jidx, NPG), axis=-1, keepdims=True)
        oh = jidx == selj
        oh_slots.append(oh.astype(jnp.float32))
        work = jnp.where(oh, jnp.float32(1e30), work)
    onehot = jnp.concatenate(oh_slots, axis=1)

    def edge_conv(feat, w1_ref, b1_ref, w2_ref, b2_ref, act):
        F = feat.shape[-1]
        H2 = w2_ref.shape[-1]
        xj = lax.dot_general(onehot, feat, (((2,), (1,)), ((0,), (0,))),
                             precision=lax.Precision.HIGHEST)
        xj = xj.reshape(G, KNN, NPG, F)
        xi = jnp.broadcast_to(feat.reshape(G, 1, NPG, F), (G, KNN, NPG, F))
        e = jnp.concatenate([xi, xj - xi], axis=-1).reshape(G * KNN * NPG, 2 * F)
        h = act(jnp.dot(e, w1_ref[...]) + b1_ref[...])
        h = act(jnp.dot(h, w2_ref[...]) + b2_ref[...])
        return jnp.max(h.reshape(G, KNN, NPG, H2), axis=1)

    x1 = edge_conv(xx, w11_ref, b11_ref, w21_ref, b21_ref, _relu)
    x2 = edge_conv(x1, w12_ref, b12_ref, w22_ref, b22_ref, _leaky)
    comb_ref[...] = jnp.concatenate([xx, x1, x2], axis=-1)


def _mlp_body(comb_ref,
              w1_ref, b1_ref, w2_ref, b2_ref, w3_ref, b3_ref,
              w4_ref, b4_ref, w5_ref, b5_ref, out_ref):
    h = _relu(jnp.dot(comb_ref[...], w1_ref[...]) + b1_ref[...])
    h = _relu(jnp.dot(h, w2_ref[...]) + b2_ref[...])
    h = _relu(jnp.dot(h, w3_ref[...]) + b3_ref[...])
    h = _relu(jnp.dot(h, w4_ref[...]) + b4_ref[...])
    out_ref[...] = jnp.dot(h, w5_ref[...]) + b5_ref[...]


def _block_diag(a, b):
    fi_a, fo_a = a.shape
    fi_b, fo_b = b.shape
    top = jnp.concatenate([a, jnp.zeros((fi_a, fo_b), a.dtype)], axis=1)
    bot = jnp.concatenate([jnp.zeros((fi_b, fo_a), b.dtype), b], axis=1)
    return jnp.concatenate([top, bot], axis=0)


@jax.jit
def kernel(x, pos, tq, batch, params):
    del batch
    N = x.shape[0]
    B = N // NPG
    p = params

    xx = jnp.concatenate([tq, x, pos], axis=1).reshape(B, NPG, FEA_IN)

    G = 4 if B % 4 == 0 else 1
    const2 = lambda i: (0, 0)
    conv_specs = [
        pl.BlockSpec((G, NPG, FEA_IN), lambda i: (i, 0, 0)),
        pl.BlockSpec((2 * FEA_IN, 32), const2),
        pl.BlockSpec((1, 32), const2),
        pl.BlockSpec((32, 32), const2),
        pl.BlockSpec((1, 32), const2),
        pl.BlockSpec((64, 64), const2),
        pl.BlockSpec((1, 64), const2),
        pl.BlockSpec((64, 32), const2),
        pl.BlockSpec((1, 32), const2),
    ]
    comb = pl.pallas_call(
        functools.partial(_conv_body, G=G),
        grid=(B // G,),
        in_specs=conv_specs,
        out_specs=pl.BlockSpec((G, NPG, 64 + FEA_IN), lambda i: (i, 0, 0)),
        out_shape=jax.ShapeDtypeStruct((B, NPG, 64 + FEA_IN), jnp.float32),
    )(xx, p['c1_w1'], p['c1_b1'][None, :], p['c1_w2'], p['c1_b2'][None, :],
      p['c2_w1'], p['c2_b1'][None, :], p['c2_w2'], p['c2_b2'][None, :])

    w1 = jnp.concatenate([p['m1_w'], p['n1_w']], axis=1)
    b1 = jnp.concatenate([p['m1_b'], p['n1_b']])[None, :]
    w2 = _block_diag(p['m2_w'], p['n2_w'])
    b2 = jnp.concatenate([p['m2_b'], p['n2_b']])[None, :]
    w3 = _block_diag(p['m3_w'], p['n3_w'])
    b3 = jnp.concatenate([p['m3_b'], p['n3_b']])[None, :]
    w4 = _block_diag(p['m4_w'], p['n4_w'])
    b4 = jnp.concatenate([p['m4_b'], p['n4_b']])[None, :]
    w5 = jnp.zeros((128, 3), jnp.float32)
    w5 = w5.at[:64, :2].set(p['m5_w']).at[64:, 2:].set(p['n5_w'])
    b5 = jnp.concatenate([p['m5_b'], p['n5_b']])[None, :]

    MLP_IN = NPG * (64 + FEA_IN)
    BM = 128 if B % 128 == 0 else B
    mlp_specs = (
        [pl.BlockSpec((BM, MLP_IN), lambda i: (i, 0))] +
        [pl.BlockSpec(w.shape, const2) for w in
         (w1, b1, w2, b2, w3, b3, w4, b4, w5, b5)]
    )
    out = pl.pallas_call(
        _mlp_body,
        grid=(B // BM,),
        in_specs=mlp_specs,
        out_specs=pl.BlockSpec((BM, 3), lambda i: (i, 0)),
        out_shape=jax.ShapeDtypeStruct((B, 3), jnp.float32),
    )(comb.reshape(B, MLP_IN), w1, b1, w2, b2, w3, b3, w4, b4, w5, b5)
    return out


# speedup vs baseline: 1.1003x; 1.1003x over previous
import functools

import jax
import jax.numpy as jnp
from jax import lax
from jax.experimental import pallas as pl

NPG = 96
KNN = 16
FEA_IN = 5


def _relu(v):
    return jnp.maximum(v, 0.0)


def _leaky(v):
    return jnp.where(v >= 0, v, 0.01 * v)


def _conv_body(xx_ref, w11_ref, b11_ref, w21_ref, b21_ref,
               w12_ref, b12_ref, w22_ref, b22_ref,
               comb_ref, *, G):
    xx = xx_ref[...]

    d2 = jnp.zeros((G, NPG, NPG), jnp.float32)
    for c in range(3):
        pc = xx[:, :, 2 + c]
        diff = pc[:, :, None] - pc[:, None, :]
        d2 = d2 + diff * diff

    jidx = lax.broadcasted_iota(jnp.int32, (G, NPG, NPG), 2).astype(jnp.float32)
    work = d2
    oh_slots = []
    for _ in range(KNN):
        m = jnp.min(work, axis=-1, keepdims=True)
        eq = work == m
        selj = jnp.min(jnp.where(eq, jidx, jnp.float32(NPG)), axis=-1,
                       keepdims=True)
        oh = jidx == selj
        oh_slots.append(oh.astype(jnp.float32))
        work = jnp.where(oh, jnp.float32(1e30), work)
    onehot = jnp.concatenate(oh_slots, axis=1)

    def edge_conv(feat, w1_ref, b1_ref, w2_ref, b2_ref, act):
        F = feat.shape[-1]
        H2 = w2_ref.shape[-1]
        u = jnp.concatenate([feat, -feat], axis=-1)
        v = jnp.concatenate([jnp.zeros_like(feat), feat], axis=-1)
        vj = lax.dot_general(onehot, v, (((2,), (1,)), ((0,), (0,))),
                             precision=lax.Precision.HIGHEST)
        e = (vj.reshape(G, KNN, NPG, 2 * F) + u.reshape(G, 1, NPG, 2 * F))
        h = act(jnp.dot(e.reshape(G * KNN * NPG, 2 * F), w1_ref[...])
                + b1_ref[...])
        h = act(jnp.dot(h, w2_ref[...]) + b2_ref[...])
        return jnp.max(h.reshape(G, KNN, NPG, H2), axis=1)

    x1 = edge_conv(xx, w11_ref, b11_ref, w21_ref, b21_ref, _relu)
    x2 = edge_conv(x1, w12_ref, b12_ref, w22_ref, b22_ref, _leaky)
    comb_ref[...] = jnp.concatenate([xx, x1, x2], axis=-1)


def _mlp_body(comb_ref,
              mw1_ref, mb1_ref, mw2_ref, mb2_ref, mw3_ref, mb3_ref,
              mw4_ref, mb4_ref, mw5_ref, mb5_ref,
              nw1_ref, nb1_ref, nw2_ref, nb2_ref, nw3_ref, nb3_ref,
              nw4_ref, nb4_ref, nw5_ref, nb5_ref, out_ref):
    comb = comb_ref[...]

    def head(w1, b1, w2, b2, w3, b3, w4, b4, w5, b5):
        h = _relu(jnp.dot(comb, w1[...]) + b1[...])
        h = _relu(jnp.dot(h, w2[...]) + b2[...])
        h = _relu(jnp.dot(h, w3[...]) + b3[...])
        h = _relu(jnp.dot(h, w4[...]) + b4[...])
        return jnp.dot(h, w5[...]) + b5[...]

    o1 = head(mw1_ref, mb1_ref, mw2_ref, mb2_ref, mw3_ref, mb3_ref,
              mw4_ref, mb4_ref, mw5_ref, mb5_ref)
    o2 = head(nw1_ref, nb1_ref, nw2_ref, nb2_ref, nw3_ref, nb3_ref,
              nw4_ref, nb4_ref, nw5_ref, nb5_ref)
    out_ref[...] = jnp.concatenate([o1, o2], axis=1)


@jax.jit
def kernel(x, pos, tq, batch, params):
    del batch
    N = x.shape[0]
    B = N // NPG
    p = params

    xx = jnp.concatenate([tq, x, pos], axis=1).reshape(B, NPG, FEA_IN)

    G = 4 if B % 4 == 0 else 1
    const2 = lambda i: (0, 0)
    conv_specs = [
        pl.BlockSpec((G, NPG, FEA_IN), lambda i: (i, 0, 0)),
        pl.BlockSpec((2 * FEA_IN, 32), const2),
        pl.BlockSpec((1, 32), const2),
        pl.BlockSpec((32, 32), const2),
        pl.BlockSpec((1, 32), const2),
        pl.BlockSpec((64, 64), const2),
        pl.BlockSpec((1, 64), const2),
        pl.BlockSpec((64, 32), const2),
        pl.BlockSpec((1, 32), const2),
    ]
    comb = pl.pallas_call(
        functools.partial(_conv_body, G=G),
        grid=(B // G,),
        in_specs=conv_specs,
        out_specs=pl.BlockSpec((G, NPG, 64 + FEA_IN), lambda i: (i, 0, 0)),
        out_shape=jax.ShapeDtypeStruct((B, NPG, 64 + FEA_IN), jnp.float32),
    )(xx, p['c1_w1'], p['c1_b1'][None, :], p['c1_w2'], p['c1_b2'][None, :],
      p['c2_w1'], p['c2_b1'][None, :], p['c2_w2'], p['c2_b2'][None, :])

    mlp_ws = []
    for pre in ('m', 'n'):
        for i in range(1, 6):
            mlp_ws.append(p['%s%d_w' % (pre, i)])
            mlp_ws.append(p['%s%d_b' % (pre, i)][None, :])

    MLP_IN = NPG * (64 + FEA_IN)
    BM = 128 if B % 128 == 0 else B
    mlp_specs = (
        [pl.BlockSpec((BM, MLP_IN), lambda i: (i, 0))] +
        [pl.BlockSpec(w.shape, const2) for w in mlp_ws]
    )
    out = pl.pallas_call(
        _mlp_body,
        grid=(B // BM,),
        in_specs=mlp_specs,
        out_specs=pl.BlockSpec((BM, 3), lambda i: (i, 0)),
        out_shape=jax.ShapeDtypeStruct((B, 3), jnp.float32),
    )(comb.reshape(B, MLP_IN), *mlp_ws)
    return out


# speedup vs baseline: 1.1696x; 1.0630x over previous
import functools

import jax
import jax.numpy as jnp
from jax import lax
from jax.experimental import pallas as pl
from jax.experimental.pallas import tpu as pltpu

NPG = 96
KNN = 16
FEA_IN = 5


def _relu(v):
    return jnp.maximum(v, 0.0)


def _leaky(v):
    return jnp.where(v >= 0, v, 0.01 * v)


def _conv_body(xx_ref, w11_ref, b11_ref, w21_ref, b21_ref,
               w12_ref, b12_ref, w22_ref, b22_ref,
               comb_ref, oh_ref, *, G):
    xx = xx_ref[...]

    d2 = jnp.zeros((G, NPG, NPG), jnp.float32)
    for c in range(3):
        pc = xx[:, :, 2 + c]
        diff = pc[:, :, None] - pc[:, None, :]
        d2 = d2 + diff * diff

    jidx = lax.broadcasted_iota(jnp.int32, (G, NPG, NPG), 2).astype(jnp.float32)
    work = d2
    for k in range(KNN):
        m = jnp.min(work, axis=-1, keepdims=True)
        eq = work == m
        selj = jnp.min(jnp.where(eq, jidx, jnp.float32(NPG)), axis=-1,
                       keepdims=True)
        oh = jidx == selj
        oh_ref[:, pl.ds(k * NPG, NPG), :] = oh.astype(jnp.float32)
        work = jnp.where(oh, jnp.float32(1e30), work)
    onehot = oh_ref[...]

    def edge_conv(feat, w1_ref, b1_ref, w2_ref, b2_ref, act):
        F = feat.shape[-1]
        H2 = w2_ref.shape[-1]
        u = jnp.concatenate([feat, -feat], axis=-1)
        v = jnp.concatenate([jnp.zeros_like(feat), feat], axis=-1)
        vj = lax.dot_general(onehot, v, (((2,), (1,)), ((0,), (0,))),
                             precision=lax.Precision.HIGHEST)
        e = (vj.reshape(G, KNN, NPG, 2 * F) + u.reshape(G, 1, NPG, 2 * F))
        h = act(jnp.dot(e.reshape(G * KNN * NPG, 2 * F), w1_ref[...])
                + b1_ref[...])
        h = act(jnp.dot(h, w2_ref[...]) + b2_ref[...])
        return jnp.max(h.reshape(G, KNN, NPG, H2), axis=1)

    x1 = edge_conv(xx, w11_ref, b11_ref, w21_ref, b21_ref, _relu)
    x2 = edge_conv(x1, w12_ref, b12_ref, w22_ref, b22_ref, _leaky)
    comb_ref[...] = jnp.concatenate([xx, x1, x2], axis=-1)


def _mlp_body(comb_ref,
              mw1_ref, mb1_ref, mw2_ref, mb2_ref, mw3_ref, mb3_ref,
              mw4_ref, mb4_ref, mw5_ref, mb5_ref,
              nw1_ref, nb1_ref, nw2_ref, nb2_ref, nw3_ref, nb3_ref,
              nw4_ref, nb4_ref, nw5_ref, nb5_ref, out_ref):
    comb = comb_ref[...]

    def head(w1, b1, w2, b2, w3, b3, w4, b4, w5, b5):
        h = _relu(jnp.dot(comb, w1[...]) + b1[...])
        h = _relu(jnp.dot(h, w2[...]) + b2[...])
        h = _relu(jnp.dot(h, w3[...]) + b3[...])
        h = _relu(jnp.dot(h, w4[...]) + b4[...])
        return jnp.dot(h, w5[...]) + b5[...]

    o1 = head(mw1_ref, mb1_ref, mw2_ref, mb2_ref, mw3_ref, mb3_ref,
              mw4_ref, mb4_ref, mw5_ref, mb5_ref)
    o2 = head(nw1_ref, nb1_ref, nw2_ref, nb2_ref, nw3_ref, nb3_ref,
              nw4_ref, nb4_ref, nw5_ref, nb5_ref)
    out_ref[...] = jnp.concatenate([o1, o2], axis=1)


@jax.jit
def kernel(x, pos, tq, batch, params):
    del batch
    N = x.shape[0]
    B = N // NPG
    p = params

    xx = jnp.concatenate([tq, x, pos], axis=1).reshape(B, NPG, FEA_IN)

    G = 8 if B % 8 == 0 else 1
    const2 = lambda i: (0, 0)
    conv_specs = [
        pl.BlockSpec((G, NPG, FEA_IN), lambda i: (i, 0, 0)),
        pl.BlockSpec((2 * FEA_IN, 32), const2),
        pl.BlockSpec((1, 32), const2),
        pl.BlockSpec((32, 32), const2),
        pl.BlockSpec((1, 32), const2),
        pl.BlockSpec((64, 64), const2),
        pl.BlockSpec((1, 64), const2),
        pl.BlockSpec((64, 32), const2),
        pl.BlockSpec((1, 32), const2),
    ]
    comb = pl.pallas_call(
        functools.partial(_conv_body, G=G),
        grid=(B // G,),
        in_specs=conv_specs,
        out_specs=pl.BlockSpec((G, NPG, 64 + FEA_IN), lambda i: (i, 0, 0)),
        out_shape=jax.ShapeDtypeStruct((B, NPG, 64 + FEA_IN), jnp.float32),
        scratch_shapes=[pltpu.VMEM((G, KNN * NPG, NPG), jnp.float32)],
    )(xx, p['c1_w1'], p['c1_b1'][None, :], p['c1_w2'], p['c1_b2'][None, :],
      p['c2_w1'], p['c2_b1'][None, :], p['c2_w2'], p['c2_b2'][None, :])

    mlp_ws = []
    for pre in ('m', 'n'):
        for i in range(1, 6):
            mlp_ws.append(p['%s%d_w' % (pre, i)])
            mlp_ws.append(p['%s%d_b' % (pre, i)][None, :])

    MLP_IN = NPG * (64 + FEA_IN)
    BM = 128 if B % 128 == 0 else B
    mlp_specs = (
        [pl.BlockSpec((BM, MLP_IN), lambda i: (i, 0))] +
        [pl.BlockSpec(w.shape, const2) for w in mlp_ws]
    )
    out = pl.pallas_call(
        _mlp_body,
        grid=(B // BM,),
        in_specs=mlp_specs,
        out_specs=pl.BlockSpec((BM, 3), lambda i: (i, 0)),
        out_shape=jax.ShapeDtypeStruct((B, 3), jnp.float32),
    )(comb.reshape(B, MLP_IN), *mlp_ws)
    return out


# speedup vs baseline: 1.1717x; 1.0018x over previous
import functools

import jax
import jax.numpy as jnp
from jax import lax
from jax.experimental import pallas as pl
from jax.experimental.pallas import tpu as pltpu

NPG = 96
KNN = 16
FEA_IN = 5


def _relu(v):
    return jnp.maximum(v, 0.0)


def _leaky(v):
    return jnp.where(v >= 0, v, 0.01 * v)


def _conv_body(xx_ref, jidx_ref, w11_ref, b11_ref, w21_ref, b21_ref,
               w12_ref, b12_ref, w22_ref, b22_ref,
               comb_ref, oh_ref, *, G):
    xx = xx_ref[...]

    d2 = jnp.zeros((G, NPG, NPG), jnp.float32)
    for c in range(3):
        pc = xx[:, :, 2 + c]
        diff = pc[:, :, None] - pc[:, None, :]
        d2 = d2 + diff * diff

    jidx = jnp.broadcast_to(jidx_ref[...], (G, NPG, NPG))
    work = d2
    for k in range(KNN):
        m = jnp.min(work, axis=-1, keepdims=True)
        eq = work == m
        selj = jnp.min(jnp.where(eq, jidx, jnp.float32(NPG)), axis=-1,
                       keepdims=True)
        oh = jidx == selj
        oh_ref[:, pl.ds(k * NPG, NPG), :] = oh.astype(jnp.float32)
        work = jnp.where(oh, jnp.float32(1e30), work)
    onehot = oh_ref[...]

    def edge_conv(feat, w1_ref, b1_ref, w2_ref, b2_ref, act):
        F = feat.shape[-1]
        H2 = w2_ref.shape[-1]
        u = jnp.concatenate([feat, -feat], axis=-1)
        v = jnp.concatenate([jnp.zeros_like(feat), feat], axis=-1)
        vj = lax.dot_general(onehot, v, (((2,), (1,)), ((0,), (0,))),
                             precision=lax.Precision.HIGHEST)
        e = (vj.reshape(G, KNN, NPG, 2 * F) + u.reshape(G, 1, NPG, 2 * F))
        h = act(jnp.dot(e.reshape(G * KNN * NPG, 2 * F), w1_ref[...])
                + b1_ref[...])
        h = act(jnp.dot(h, w2_ref[...]) + b2_ref[...])
        return jnp.max(h.reshape(G, KNN, NPG, H2), axis=1)

    x1 = edge_conv(xx, w11_ref, b11_ref, w21_ref, b21_ref, _relu)
    x2 = edge_conv(x1, w12_ref, b12_ref, w22_ref, b22_ref, _leaky)
    comb_ref[...] = jnp.concatenate([xx, x1, x2], axis=-1)


def _mlp_body(comb_ref,
              mw1_ref, mb1_ref, mw2_ref, mb2_ref, mw3_ref, mb3_ref,
              mw4_ref, mb4_ref, mw5_ref, mb5_ref,
              nw1_ref, nb1_ref, nw2_ref, nb2_ref, nw3_ref, nb3_ref,
              nw4_ref, nb4_ref, nw5_ref, nb5_ref, out_ref):
    comb = comb_ref[...]

    def head(w1, b1, w2, b2, w3, b3, w4, b4, w5, b5):
        h = _relu(jnp.dot(comb, w1[...]) + b1[...])
        h = _relu(jnp.dot(h, w2[...]) + b2[...])
        h = _relu(jnp.dot(h, w3[...]) + b3[...])
        h = _relu(jnp.dot(h, w4[...]) + b4[...])
        return jnp.dot(h, w5[...]) + b5[...]

    o1 = head(mw1_ref, mb1_ref, mw2_ref, mb2_ref, mw3_ref, mb3_ref,
              mw4_ref, mb4_ref, mw5_ref, mb5_ref)
    o2 = head(nw1_ref, nb1_ref, nw2_ref, nb2_ref, nw3_ref, nb3_ref,
              nw4_ref, nb4_ref, nw5_ref, nb5_ref)
    out_ref[...] = jnp.concatenate([o1, o2], axis=1)


@jax.jit
def kernel(x, pos, tq, batch, params):
    del batch
    N = x.shape[0]
    B = N // NPG
    p = params

    xx = jnp.concatenate([tq, x, pos], axis=1).reshape(B, NPG, FEA_IN)

    G = 8 if B % 8 == 0 else 1
    const2 = lambda i: (0, 0)
    jidx_host = jnp.broadcast_to(
        jnp.arange(NPG, dtype=jnp.float32)[None, None, :], (1, NPG, NPG))
    conv_specs = [
        pl.BlockSpec((G, NPG, FEA_IN), lambda i: (i, 0, 0)),
        pl.BlockSpec((1, NPG, NPG), lambda i: (0, 0, 0)),
        pl.BlockSpec((2 * FEA_IN, 32), const2),
        pl.BlockSpec((1, 32), const2),
        pl.BlockSpec((32, 32), const2),
        pl.BlockSpec((1, 32), const2),
        pl.BlockSpec((64, 64), const2),
        pl.BlockSpec((1, 64), const2),
        pl.BlockSpec((64, 32), const2),
        pl.BlockSpec((1, 32), const2),
    ]
    comb = pl.pallas_call(
        functools.partial(_conv_body, G=G),
        grid=(B // G,),
        in_specs=conv_specs,
        out_specs=pl.BlockSpec((G, NPG, 64 + FEA_IN), lambda i: (i, 0, 0)),
        out_shape=jax.ShapeDtypeStruct((B, NPG, 64 + FEA_IN), jnp.float32),
        scratch_shapes=[pltpu.VMEM((G, KNN * NPG, NPG), jnp.float32)],
    )(xx, jidx_host, p['c1_w1'], p['c1_b1'][None, :], p['c1_w2'],
      p['c1_b2'][None, :],
      p['c2_w1'], p['c2_b1'][None, :], p['c2_w2'], p['c2_b2'][None, :])

    mlp_ws = []
    for pre in ('m', 'n'):
        for i in range(1, 6):
            mlp_ws.append(p['%s%d_w' % (pre, i)])
            mlp_ws.append(p['%s%d_b' % (pre, i)][None, :])

    MLP_IN = NPG * (64 + FEA_IN)
    BM = 128 if B % 128 == 0 else B
    mlp_specs = (
        [pl.BlockSpec((BM, MLP_IN), lambda i: (i, 0))] +
        [pl.BlockSpec(w.shape, const2) for w in mlp_ws]
    )
    out = pl.pallas_call(
        _mlp_body,
        grid=(B // BM,),
        in_specs=mlp_specs,
        out_specs=pl.BlockSpec((BM, 3), lambda i: (i, 0)),
        out_shape=jax.ShapeDtypeStruct((B, 3), jnp.float32),
    )(comb.reshape(B, MLP_IN), *mlp_ws)
    return out


# speedup vs baseline: 1.2076x; 1.0306x over previous
import functools

import jax
import jax.numpy as jnp
from jax import lax
from jax.experimental import pallas as pl
from jax.experimental.pallas import tpu as pltpu

NPG = 96
KNN = 16
FEA_IN = 5


def _relu(v):
    return jnp.maximum(v, 0.0)


def _leaky(v):
    return jnp.where(v >= 0, v, 0.01 * v)


def _conv_body(xx_ref, jidx_ref, w11_ref, b11_ref, w21_ref, b21_ref,
               w12_ref, b12_ref, w22_ref, b22_ref,
               comb_ref, oh_ref, *, G):
    xx = xx_ref[...]

    d2 = jnp.zeros((G, NPG, NPG), jnp.float32)
    for c in range(3):
        pc = xx[:, :, 2 + c]
        diff = pc[:, :, None] - pc[:, None, :]
        d2 = d2 + diff * diff

    jidx = jnp.broadcast_to(jidx_ref[...], (G, NPG, NPG))
    @pl.when(pl.program_id(0) == 0)
    def _init():
        oh_ref[...] = jnp.zeros(oh_ref.shape, jnp.float32)

    work = d2
    for k in range(KNN):
        m = jnp.min(work, axis=-1, keepdims=True)
        eq = work == m
        selj = jnp.min(jnp.where(eq, jidx, jnp.float32(NPG)), axis=-1,
                       keepdims=True)
        oh = jidx == selj
        oh_ref[:, pl.ds((k // 2) * NPG, NPG),
               pl.ds((k % 2) * 128, NPG)] = oh.astype(jnp.float32)
        work = jnp.where(oh, jnp.float32(1e30), work)
    onehot = oh_ref[...]

    def edge_conv(feat, w1_ref, b1_ref, w2_ref, b2_ref, act):
        F = feat.shape[-1]
        H2 = w2_ref.shape[-1] // 2
        u2 = jnp.concatenate([feat, -feat, feat, -feat], axis=-1)
        v = jnp.concatenate([jnp.zeros_like(feat), feat], axis=-1)
        vpad = jnp.concatenate(
            [v, jnp.zeros((G, 128 - NPG, 2 * F), jnp.float32)], axis=1)
        z = jnp.zeros_like(vpad)
        v2 = jnp.concatenate(
            [jnp.concatenate([vpad, z], axis=-1),
             jnp.concatenate([z, vpad], axis=-1)], axis=1)
        vj = lax.dot_general(onehot, v2, (((2,), (1,)), ((0,), (0,))),
                             precision=lax.Precision.HIGHEST)
        e = (vj.reshape(G, KNN // 2, NPG, 4 * F)
             + u2.reshape(G, 1, NPG, 4 * F))
        h = act(jnp.dot(e.reshape(G * KNN // 2 * NPG, 4 * F), w1_ref[...])
                + b1_ref[...])
        h = act(jnp.dot(h, w2_ref[...]) + b2_ref[...])
        x = jnp.max(h.reshape(G, KNN // 2, NPG, 2 * H2), axis=1)
        return jnp.maximum(x[..., :H2], x[..., H2:])

    x1 = edge_conv(xx, w11_ref, b11_ref, w21_ref, b21_ref, _relu)
    x2 = edge_conv(x1, w12_ref, b12_ref, w22_ref, b22_ref, _leaky)
    comb_ref[...] = jnp.concatenate([xx, x1, x2], axis=-1)


def _mlp_body(comb_ref,
              mw1_ref, mb1_ref, mw2_ref, mb2_ref, mw3_ref, mb3_ref,
              mw4_ref, mb4_ref, mw5_ref, mb5_ref,
              nw1_ref, nb1_ref, nw2_ref, nb2_ref, nw3_ref, nb3_ref,
              nw4_ref, nb4_ref, nw5_ref, nb5_ref, out_ref):
    comb = comb_ref[...]

    def head(w1, b1, w2, b2, w3, b3, w4, b4, w5, b5):
        h = _relu(jnp.dot(comb, w1[...]) + b1[...])
        h = _relu(jnp.dot(h, w2[...]) + b2[...])
        h = _relu(jnp.dot(h, w3[...]) + b3[...])
        h = _relu(jnp.dot(h, w4[...]) + b4[...])
        return jnp.dot(h, w5[...]) + b5[...]

    o1 = head(mw1_ref, mb1_ref, mw2_ref, mb2_ref, mw3_ref, mb3_ref,
              mw4_ref, mb4_ref, mw5_ref, mb5_ref)
    o2 = head(nw1_ref, nb1_ref, nw2_ref, nb2_ref, nw3_ref, nb3_ref,
              nw4_ref, nb4_ref, nw5_ref, nb5_ref)
    out_ref[...] = jnp.concatenate([o1, o2], axis=1)


@jax.jit
def kernel(x, pos, tq, batch, params):
    del batch
    N = x.shape[0]
    B = N // NPG
    p = params

    xx = jnp.concatenate([tq, x, pos], axis=1).reshape(B, NPG, FEA_IN)

    G = 8 if B % 8 == 0 else 1
    const2 = lambda i: (0, 0)
    jidx_host = jnp.broadcast_to(
        jnp.arange(NPG, dtype=jnp.float32)[None, None, :], (1, NPG, NPG))

    def dbl(w):
        z = jnp.zeros_like(w)
        return jnp.concatenate([jnp.concatenate([w, z], axis=1),
                                jnp.concatenate([z, w], axis=1)], axis=0)

    c1w1, c1b1 = dbl(p['c1_w1']), jnp.tile(p['c1_b1'], 2)[None, :]
    c1w2, c1b2 = dbl(p['c1_w2']), jnp.tile(p['c1_b2'], 2)[None, :]
    c2w1, c2b1 = dbl(p['c2_w1']), jnp.tile(p['c2_b1'], 2)[None, :]
    c2w2, c2b2 = dbl(p['c2_w2']), jnp.tile(p['c2_b2'], 2)[None, :]
    conv_specs = [
        pl.BlockSpec((G, NPG, FEA_IN), lambda i: (i, 0, 0)),
        pl.BlockSpec((1, NPG, NPG), lambda i: (0, 0, 0)),
        pl.BlockSpec((4 * FEA_IN, 64), const2),
        pl.BlockSpec((1, 64), const2),
        pl.BlockSpec((64, 64), const2),
        pl.BlockSpec((1, 64), const2),
        pl.BlockSpec((128, 128), const2),
        pl.BlockSpec((1, 128), const2),
        pl.BlockSpec((128, 64), const2),
        pl.BlockSpec((1, 64), const2),
    ]
    comb = pl.pallas_call(
        functools.partial(_conv_body, G=G),
        grid=(B // G,),
        in_specs=conv_specs,
        out_specs=pl.BlockSpec((G, NPG, 64 + FEA_IN), lambda i: (i, 0, 0)),
        out_shape=jax.ShapeDtypeStruct((B, NPG, 64 + FEA_IN), jnp.float32),
        scratch_shapes=[pltpu.VMEM((G, KNN // 2 * NPG, 256), jnp.float32)],
    )(xx, jidx_host, c1w1, c1b1, c1w2, c1b2, c2w1, c2b1, c2w2, c2b2)

    mlp_ws = []
    for pre in ('m', 'n'):
        for i in range(1, 6):
            mlp_ws.append(p['%s%d_w' % (pre, i)])
            mlp_ws.append(p['%s%d_b' % (pre, i)][None, :])

    MLP_IN = NPG * (64 + FEA_IN)
    BM = 128 if B % 128 == 0 else B
    mlp_specs = (
        [pl.BlockSpec((BM, MLP_IN), lambda i: (i, 0))] +
        [pl.BlockSpec(w.shape, const2) for w in mlp_ws]
    )
    out = pl.pallas_call(
        _mlp_body,
        grid=(B // BM,),
        in_specs=mlp_specs,
        out_specs=pl.BlockSpec((BM, 3), lambda i: (i, 0)),
        out_shape=jax.ShapeDtypeStruct((B, 3), jnp.float32),
    )(comb.reshape(B, MLP_IN), *mlp_ws)
    return out
